# baseline (speedup 1.0000x reference)
import jax
import jax.numpy as jnp
from jax import lax
from jax.experimental import pallas as pl
from jax.experimental.pallas import tpu as pltpu
from jax.experimental.pallas import tpu_sc as plsc

RES = 50
NC, NS, L = 2, 16, 16
NW = NC * NS
CHUNK = 4096
BLK = 128
CB_PAD = 2560


def _sc_body(n_per_w, pts_hbm, cb_hbm, out_hbm, cb_v, in_v0, in_v1,
             out_v0, out_v1, sem_cb, sem_in, sem_out):
    wid = lax.axis_index("s") * NC + lax.axis_index("c")
    base = wid * n_per_w
    n_chunks = n_per_w // CHUNK
    in_bufs = (in_v0, in_v1)
    out_bufs = (out_v0, out_v1)

    def in_copy(c, buf):
        off = base + c * CHUNK
        return pltpu.make_async_copy(
            pts_hbm.at[pl.ds(2 * off, 2 * CHUNK)], in_bufs[buf], sem_in)

    def out_copy(c, buf):
        off = base + c * CHUNK
        return pltpu.make_async_copy(
            out_bufs[buf], out_hbm.at[pl.ds(off, CHUNK)], sem_out)

    cb_copy = pltpu.make_async_copy(cb_hbm, cb_v, sem_cb)
    cb_copy.start()
    in_copy(0, 0).start()
    cb_copy.wait()

    for c in range(n_chunks):
        buf = c % 2
        if c + 1 < n_chunks:
            in_copy(c + 1, 1 - buf).start()
        in_copy(c, buf).wait()
        if c >= 2:
            out_copy(c - 2, buf).wait()
        ib = in_bufs[buf]
        ob = out_bufs[buf]

        @plsc.parallel_loop(0, CHUNK // BLK, unroll=4)
        def _blk(b):
            for k in range(BLK // L):
                x = ib[pl.ds(b * 2 * BLK + k * L, L)]
                y = ib[pl.ds(b * 2 * BLK + BLK + k * L, L)]
                xi = (x * 49.0).astype(jnp.int32)
                yi = (y * 49.0).astype(jnp.int32)
                cidx = xi + yi * RES
                ob[pl.ds(b * BLK + k * L, L)] = plsc.load_gather(cb_v, [cidx])

        out_copy(c, buf).start()

    if n_chunks >= 2:
        out_copy(n_chunks - 2, n_chunks % 2).wait()
    out_copy(n_chunks - 1, (n_chunks - 1) % 2).wait()


def kernel(pts, codebook_0):
    n = pts.shape[0]
    n_per_w = n // NW
    pts_flat = pts.reshape(n // BLK, BLK, 2).transpose(0, 2, 1).reshape(-1)
    cb_flat = jnp.pad(codebook_0.reshape(-1),
                      (0, CB_PAD - codebook_0.shape[0]))
    mesh = plsc.VectorSubcoreMesh(core_axis_name="c", subcore_axis_name="s")
    run = pl.kernel(
        lambda *refs: _sc_body(n_per_w, *refs),
        out_type=jax.ShapeDtypeStruct((n,), jnp.float32),
        mesh=mesh,
        scratch_types=[
            pltpu.VMEM((CB_PAD,), jnp.float32),
            pltpu.VMEM((2 * CHUNK,), jnp.float32),
            pltpu.VMEM((2 * CHUNK,), jnp.float32),
            pltpu.VMEM((CHUNK,), jnp.float32),
            pltpu.VMEM((CHUNK,), jnp.float32),
            pltpu.SemaphoreType.DMA,
            pltpu.SemaphoreType.DMA,
            pltpu.SemaphoreType.DMA,
        ],
        compiler_params=pltpu.CompilerParams(needs_layout_passes=False),
    )
    out = run(pts_flat, cb_flat)
    return out.reshape(n, 1)


# speedup vs baseline: 1.1707x; 1.1707x over previous
import jax
import jax.numpy as jnp
from jax import lax
from jax.experimental import pallas as pl
from jax.experimental.pallas import tpu as pltpu
from jax.experimental.pallas import tpu_sc as plsc

RES = 50
NC, NS, L = 2, 16, 16
NW = NC * NS
CHUNK = 8192
BLK = 128
CB_PAD = 2560


def _sc_body(n_per_w, pts_hbm, cb_hbm, out_hbm, cb_v, in_v0, in_v1,
             out_v0, out_v1, sem_cb, sem_in, sem_out):
    wid = lax.axis_index("s") * NC + lax.axis_index("c")
    base = wid * n_per_w
    n_chunks = n_per_w // CHUNK
    in_bufs = (in_v0, in_v1)
    out_bufs = (out_v0, out_v1)

    def in_copy(c, buf):
        off = base + c * CHUNK
        return pltpu.make_async_copy(
            pts_hbm.at[pl.ds(2 * off, 2 * CHUNK)], in_bufs[buf], sem_in)

    def out_copy(c, buf):
        off = base + c * CHUNK
        return pltpu.make_async_copy(
            out_bufs[buf], out_hbm.at[pl.ds(off, CHUNK)], sem_out)

    cb_copy = pltpu.make_async_copy(cb_hbm, cb_v, sem_cb)
    cb_copy.start()
    in_copy(0, 0).start()
    cb_copy.wait()

    for c in range(n_chunks):
        buf = c % 2
        if c + 1 < n_chunks:
            in_copy(c + 1, 1 - buf).start()
        in_copy(c, buf).wait()
        if c >= 2:
            out_copy(c - 2, buf).wait()
        ib = in_bufs[buf]
        ob = out_bufs[buf]

        @plsc.parallel_loop(0, CHUNK // BLK, unroll=4)
        def _blk(b):
            for k in range(BLK // L):
                x = ib[pl.ds(b * 2 * BLK + k * L, L)]
                y = ib[pl.ds(b * 2 * BLK + BLK + k * L, L)]
                xi = (x * 49.0).astype(jnp.int32)
                yi = (y * 49.0).astype(jnp.int32)
                cidx = xi + yi * RES
                ob[pl.ds(b * BLK + k * L, L)] = plsc.load_gather(cb_v, [cidx])

        out_copy(c, buf).start()

    if n_chunks >= 2:
        out_copy(n_chunks - 2, n_chunks % 2).wait()
    out_copy(n_chunks - 1, (n_chunks - 1) % 2).wait()


def kernel(pts, codebook_0):
    n = pts.shape[0]
    n_per_w = n // NW
    pts_flat = pts.reshape(n // BLK, BLK, 2).transpose(0, 2, 1).reshape(-1)
    cb_flat = jnp.pad(codebook_0.reshape(-1),
                      (0, CB_PAD - codebook_0.shape[0]))
    mesh = plsc.VectorSubcoreMesh(core_axis_name="c", subcore_axis_name="s")
    run = pl.kernel(
        lambda *refs: _sc_body(n_per_w, *refs),
        out_type=jax.ShapeDtypeStruct((n,), jnp.float32),
        mesh=mesh,
        scratch_types=[
            pltpu.VMEM((CB_PAD,), jnp.float32),
            pltpu.VMEM((2 * CHUNK,), jnp.float32),
            pltpu.VMEM((2 * CHUNK,), jnp.float32),
            pltpu.VMEM((CHUNK,), jnp.float32),
            pltpu.VMEM((CHUNK,), jnp.float32),
            pltpu.SemaphoreType.DMA,
            pltpu.SemaphoreType.DMA,
            pltpu.SemaphoreType.DMA,
        ],
        compiler_params=pltpu.CompilerParams(needs_layout_passes=False),
    )
    out = run(pts_flat, cb_flat)
    return out.reshape(n, 1)
